# baseline (speedup 1.0000x reference)
import functools

import jax
import jax.numpy as jnp
from jax import lax
from jax.experimental import pallas as pl

_BLOCK = 256


def _moe_router_kernel(x_ref, w_ref, zeros_ref, logits_ref):
    zeros_ref[...] = jnp.zeros_like(zeros_ref)
    logits_ref[...] = lax.dot_general(
        w_ref[...],
        x_ref[...],
        (((1,), (1,)), ((), ())),
        preferred_element_type=jnp.float32,
    )


@functools.partial(jax.jit, static_argnames=())
def kernel(hidden_states, w_gate):
    batch, seq, hidden = hidden_states.shape
    x = hidden_states.reshape(-1, hidden)
    tokens = x.shape[0]
    experts = w_gate.shape[0]

    grid = (tokens // _BLOCK,)
    seq_blocks = seq // _BLOCK
    zeros3d, logits_t = pl.pallas_call(
        _moe_router_kernel,
        grid=grid,
        in_specs=[
            pl.BlockSpec((_BLOCK, hidden), lambda i: (i, 0)),
            pl.BlockSpec((experts, hidden), lambda i: (0, 0)),
        ],
        out_specs=[
            pl.BlockSpec(
                (1, _BLOCK, hidden),
                lambda i: (i // seq_blocks, i % seq_blocks, 0),
            ),
            pl.BlockSpec((experts, _BLOCK), lambda i: (0, i)),
        ],
        out_shape=[
            jax.ShapeDtypeStruct((batch, seq, hidden), hidden_states.dtype),
            jax.ShapeDtypeStruct((experts, tokens), jnp.float32),
        ],
    )(x, w_gate)
    return zeros3d, logits_t.T


# speedup vs baseline: 1.0272x; 1.0272x over previous
import functools

import jax
import jax.numpy as jnp
from jax import lax
from jax.experimental import pallas as pl

_BLOCK = 512
_ZBLOCK = 1024


def _moe_router_kernel(x_ref, w_ref, zeros_ref, logits_ref):
    i = pl.program_id(0)

    @pl.when(i % (_ZBLOCK // _BLOCK) == 0)
    def _():
        zeros_ref[...] = jnp.zeros_like(zeros_ref)

    logits_ref[...] = lax.dot_general(
        w_ref[...],
        x_ref[...],
        (((1,), (1,)), ((), ())),
        preferred_element_type=jnp.float32,
    )


@functools.partial(jax.jit, static_argnames=())
def kernel(hidden_states, w_gate):
    batch, seq, hidden = hidden_states.shape
    x = hidden_states.reshape(-1, hidden)
    tokens = x.shape[0]
    experts = w_gate.shape[0]

    grid = (tokens // _BLOCK,)
    r = _ZBLOCK // _BLOCK
    zseq = seq // _ZBLOCK
    zeros3d, logits_t = pl.pallas_call(
        _moe_router_kernel,
        grid=grid,
        in_specs=[
            pl.BlockSpec((_BLOCK, hidden), lambda i: (i, 0)),
            pl.BlockSpec((experts, hidden), lambda i: (0, 0)),
        ],
        out_specs=[
            pl.BlockSpec(
                (1, _ZBLOCK, hidden),
                lambda i: ((i // r) // zseq, (i // r) % zseq, 0),
            ),
            pl.BlockSpec((experts, _BLOCK), lambda i: (0, i)),
        ],
        out_shape=[
            jax.ShapeDtypeStruct((batch, seq, hidden), hidden_states.dtype),
            jax.ShapeDtypeStruct((experts, tokens), jnp.float32),
        ],
    )(x, w_gate)
    return zeros3d, logits_t.T


# speedup vs baseline: 1.0819x; 1.0533x over previous
import functools

import jax
import jax.numpy as jnp
from jax import lax
from jax.experimental import pallas as pl

_BLOCK = 512


def _moe_router_kernel(x_ref, w_ref, zeros_ref, logits_ref):
    zeros_ref[...] = jnp.zeros_like(zeros_ref)
    logits_ref[...] = lax.dot_general(
        w_ref[...],
        x_ref[...],
        (((1,), (1,)), ((), ())),
        preferred_element_type=jnp.float32,
    )


@functools.partial(jax.jit, static_argnames=())
def kernel(hidden_states, w_gate):
    batch, seq, hidden = hidden_states.shape
    x = hidden_states.reshape(-1, hidden)
    tokens = x.shape[0]
    experts = w_gate.shape[0]

    grid = (tokens // _BLOCK,)
    seq_blocks = seq // _BLOCK
    zeros3d, logits_t = pl.pallas_call(
        _moe_router_kernel,
        grid=grid,
        in_specs=[
            pl.BlockSpec((_BLOCK, hidden), lambda i: (i, 0)),
            pl.BlockSpec((experts, hidden), lambda i: (0, 0)),
        ],
        out_specs=[
            pl.BlockSpec(
                (1, _BLOCK, hidden),
                lambda i: (i // seq_blocks, i % seq_blocks, 0),
            ),
            pl.BlockSpec((experts, _BLOCK), lambda i: (0, i)),
        ],
        out_shape=[
            jax.ShapeDtypeStruct((batch, seq, hidden), hidden_states.dtype),
            jax.ShapeDtypeStruct((experts, tokens), jnp.float32),
        ],
    )(x, w_gate)
    return zeros3d, logits_t.T
